# Initial kernel scaffold; baseline (speedup 1.0000x reference)
import jax
import jax.numpy as jnp
from jax.experimental import pallas as pl


def kernel(features, edge_index, W1, b1, W2, b2, W3, b3):
    raise NotImplementedError("write your pallas kernel here")


# speedup vs baseline: 8.5598x; 8.5598x over previous
import functools

import jax
import jax.numpy as jnp
from jax import lax
from jax.experimental import pallas as pl
from jax.experimental.pallas import tpu as pltpu
from jax.experimental.pallas import tpu_sc as plsc

N = 10000
NPAD = 10240
E = 320000
CH = 128
NCHUNK = 160
EPAD = 16 * NCHUNK * CH
ROWS = NPAD // 16
SEG = 40
RB = 2048

_mesh = plsc.VectorSubcoreMesh(core_axis_name="c", subcore_axis_name="s")


def _prop_body(table, ei_hbm, out_col, e0, nchunk, sid,
               srcv, dstv, bufa, bufb, acc, sema, semb):
    r0 = sid * ROWS
    pltpu.sync_copy(table.at[pl.ds(r0, ROWS)], acc.at[pl.ds(r0, ROWS)])
    plsc.subcore_barrier()
    for g in range(nchunk // SEG):
        pltpu.sync_copy(ei_hbm.at[0, sid, pl.ds(e0 + g * SEG, SEG)], srcv)
        pltpu.sync_copy(ei_hbm.at[1, sid, pl.ds(e0 + g * SEG, SEG)], dstv)
        pltpu.async_copy(table.at[srcv.at[0]], bufa, sema)
        pltpu.async_copy(table.at[srcv.at[1]], bufb, semb)

        @pl.loop(0, SEG - 2, step=2)
        def _(j):
            pltpu.make_async_copy(table.at[srcv.at[j]], bufa, sema).wait()
            pltpu.sync_copy(bufa, acc.at[dstv.at[j]], add=True)
            pltpu.async_copy(table.at[srcv.at[j + 2]], bufa, sema)
            pltpu.make_async_copy(table.at[srcv.at[j + 1]], bufb, semb).wait()
            pltpu.sync_copy(bufb, acc.at[dstv.at[j + 1]], add=True)
            pltpu.async_copy(table.at[srcv.at[j + 3]], bufb, semb)

        pltpu.make_async_copy(table.at[srcv.at[SEG - 2]], bufa, sema).wait()
        pltpu.sync_copy(bufa, acc.at[dstv.at[SEG - 2]], add=True)
        pltpu.make_async_copy(table.at[srcv.at[SEG - 1]], bufb, semb).wait()
        pltpu.sync_copy(bufb, acc.at[dstv.at[SEG - 1]], add=True)

    plsc.subcore_barrier()
    pltpu.sync_copy(acc.at[pl.ds(r0, ROWS)], out_col.at[pl.ds(r0, ROWS)])


@functools.partial(
    pl.kernel,
    out_type=jax.ShapeDtypeStruct((2, NPAD, 128), jnp.float32),
    mesh=_mesh,
    scratch_types=[
        pltpu.VMEM((SEG, CH), jnp.int32),
        pltpu.VMEM((SEG, CH), jnp.int32),
        pltpu.VMEM((CH, 128), jnp.float32),
        pltpu.VMEM((CH, 128), jnp.float32),
        pltpu.VMEM_SHARED((NPAD, 128), jnp.float32),
        pltpu.SemaphoreType.DMA,
        pltpu.SemaphoreType.DMA,
    ],
    compiler_params=pltpu.CompilerParams(needs_layout_passes=False),
)
def _prop_es(g_hbm, ei_hbm, out_hbm, srcv, dstv, bufa, bufb, acc, sema, semb):
    cid = lax.axis_index("c")
    sid = lax.axis_index("s")
    _prop_body(g_hbm, ei_hbm, out_hbm.at[cid], cid * (NCHUNK // 2), NCHUNK // 2,
               sid, srcv, dstv, bufa, bufb, acc, sema, semb)


@functools.partial(
    pl.kernel,
    out_type=jax.ShapeDtypeStruct((2, NPAD, 128), jnp.float32),
    mesh=_mesh,
    scratch_types=[
        pltpu.VMEM((SEG, CH), jnp.int32),
        pltpu.VMEM((SEG, CH), jnp.int32),
        pltpu.VMEM((CH, 128), jnp.float32),
        pltpu.VMEM((CH, 128), jnp.float32),
        pltpu.VMEM_SHARED((NPAD, 128), jnp.float32),
        pltpu.SemaphoreType.DMA,
        pltpu.SemaphoreType.DMA,
    ],
    compiler_params=pltpu.CompilerParams(needs_layout_passes=False),
)
def _prop_fs(g_hbm, ei_hbm, out_hbm, srcv, dstv, bufa, bufb, acc, sema, semb):
    cid = lax.axis_index("c")
    sid = lax.axis_index("s")
    _prop_body(g_hbm.at[cid], ei_hbm, out_hbm.at[cid], 0, NCHUNK,
               sid, srcv, dstv, bufa, bufb, acc, sema, semb)


@functools.partial(
    pl.kernel,
    out_type=jax.ShapeDtypeStruct((2, NPAD), jnp.float32),
    mesh=_mesh,
    scratch_types=[
        pltpu.VMEM((NCHUNK // 2, CH), jnp.int32),
        pltpu.VMEM((NPAD,), jnp.float32),
        pltpu.VMEM((16, ROWS), jnp.float32),
        pltpu.VMEM_SHARED((16, NPAD), jnp.float32),
    ],
    compiler_params=pltpu.CompilerParams(needs_layout_passes=False),
)
def _sc_degree(ei_hbm, out_hbm, dstv, hist, slab, shared):
    cid = lax.axis_index("c")
    sid = lax.axis_index("s")
    pltpu.sync_copy(ei_hbm.at[1, sid, pl.ds(cid * (NCHUNK // 2), NCHUNK // 2)], dstv)

    @pl.loop(0, NPAD // 16)
    def _(i):
        hist[pl.ds(i * 16, 16)] = jnp.zeros((16,), jnp.float32)

    ones = jnp.ones((16,), jnp.float32)
    lane = lax.iota(jnp.int32, 16)

    @pl.loop(0, NCHUNK // 2)
    def _(j):
        for k in range(CH // 16):
            idx = dstv[j, pl.ds(k * 16, 16)]
            for l in range(16):
                plsc.addupdate_scatter(hist, [idx], ones, mask=lane == l)

    pltpu.sync_copy(hist, shared.at[sid])
    plsc.subcore_barrier()
    pltpu.sync_copy(shared.at[:, pl.ds(sid * ROWS, ROWS)], slab)

    @pl.loop(0, ROWS // 16)
    def _(c):
        acc16 = slab[0, pl.ds(c * 16, 16)]
        for r in range(1, 16):
            acc16 += slab[r, pl.ds(c * 16, 16)]
        hist[pl.ds(c * 16, 16)] = acc16

    pltpu.sync_copy(hist.at[pl.ds(0, ROWS)], out_hbm.at[cid, pl.ds(sid * ROWS, ROWS)])


def _dinv(deg_ref):
    return lax.rsqrt(deg_ref[0] + deg_ref[1] + 1.0)


def _tc_g0(x_pad, degp):
    def body(x_ref, deg_ref, o_ref):
        o_ref[...] = x_ref[...] * _dinv(deg_ref)

    return pl.pallas_call(
        body,
        grid=(NPAD // RB,),
        in_specs=[
            pl.BlockSpec((RB, 128), lambda i: (i, 0)),
            pl.BlockSpec((2, RB, 1), lambda i: (0, i, 0)),
        ],
        out_specs=pl.BlockSpec((RB, 128), lambda i: (i, 0)),
        out_shape=jax.ShapeDtypeStruct((NPAD, 128), jnp.float32),
    )(x_pad, degp)


def _tc_layer1(s0p, g0, degp, W1, b1):
    def body(s_ref, g_ref, deg_ref, w_ref, b_ref, o_ref):
        dinv = _dinv(deg_ref)
        u = (s_ref[0] + s_ref[1] - g_ref[...]) * dinv
        h = jnp.dot(u, w_ref[...], preferred_element_type=jnp.float32)
        g = jax.nn.relu(h + b_ref[...]) * dinv
        o_ref[0] = g[:, :128]
        o_ref[1] = g[:, 128:]

    return pl.pallas_call(
        body,
        grid=(NPAD // RB,),
        in_specs=[
            pl.BlockSpec((2, RB, 128), lambda i: (0, i, 0)),
            pl.BlockSpec((RB, 128), lambda i: (i, 0)),
            pl.BlockSpec((2, RB, 1), lambda i: (0, i, 0)),
            pl.BlockSpec((128, 256), lambda i: (0, 0)),
            pl.BlockSpec((1, 256), lambda i: (0, 0)),
        ],
        out_specs=pl.BlockSpec((2, RB, 128), lambda i: (0, i, 0)),
        out_shape=jax.ShapeDtypeStruct((2, NPAD, 128), jnp.float32),
    )(s0p, g0, degp, W1, b1)


def _tc_layer23(s1, degp, W2, b2, W3p):
    def body(s_ref, deg_ref, w2_ref, b2_ref, w3_ref, o_ref):
        dinv = _dinv(deg_ref)
        h = jnp.dot(s_ref[0] * dinv, w2_ref[:128], preferred_element_type=jnp.float32)
        h += jnp.dot(s_ref[1] * dinv, w2_ref[128:], preferred_element_type=jnp.float32)
        h = jax.nn.relu(h + b2_ref[...])
        o_ref[...] = jnp.dot(h, w3_ref[...], preferred_element_type=jnp.float32) * dinv

    return pl.pallas_call(
        body,
        grid=(NPAD // RB,),
        in_specs=[
            pl.BlockSpec((2, RB, 128), lambda i: (0, i, 0)),
            pl.BlockSpec((2, RB, 1), lambda i: (0, i, 0)),
            pl.BlockSpec((256, 256), lambda i: (0, 0)),
            pl.BlockSpec((1, 256), lambda i: (0, 0)),
            pl.BlockSpec((256, 128), lambda i: (0, 0)),
        ],
        out_specs=pl.BlockSpec((RB, 128), lambda i: (i, 0)),
        out_shape=jax.ShapeDtypeStruct((NPAD, 128), jnp.float32),
    )(s1, degp, W2, b2, W3p)


def _tc_final(s2p, g2, degp, b3):
    def body(s_ref, g_ref, deg_ref, b_ref, o_ref):
        dinv = _dinv(deg_ref)
        s = (s_ref[0] + s_ref[1] - g_ref[...]) * dinv
        o_ref[...] = s[:, :64] + b_ref[...]

    return pl.pallas_call(
        body,
        grid=(NPAD // RB,),
        in_specs=[
            pl.BlockSpec((2, RB, 128), lambda i: (0, i, 0)),
            pl.BlockSpec((RB, 128), lambda i: (i, 0)),
            pl.BlockSpec((2, RB, 1), lambda i: (0, i, 0)),
            pl.BlockSpec((1, 64), lambda i: (0, 0)),
        ],
        out_specs=pl.BlockSpec((RB, 64), lambda i: (i, 0)),
        out_shape=jax.ShapeDtypeStruct((NPAD, 64), jnp.float32),
    )(s2p, g2, degp, b3)


def kernel(features, edge_index, W1, b1, W2, b2, W3, b3):
    x_pad = jnp.pad(features, ((0, NPAD - N), (0, 0)))
    W3p = jnp.pad(W3, ((0, 0), (0, 64)))
    pad_idx = jnp.full((EPAD - E,), N, dtype=jnp.int32)
    src = jnp.concatenate([edge_index[0], pad_idx]).reshape(16, NCHUNK, CH)
    dst = jnp.concatenate([edge_index[1], pad_idx]).reshape(16, NCHUNK, CH)
    ei = jnp.stack([src, dst])

    degp = _sc_degree(ei).reshape(2, NPAD, 1)
    g0 = _tc_g0(x_pad, degp)
    s0p = _prop_es(g0, ei)
    g1 = _tc_layer1(s0p, g0, degp, W1, b1.reshape(1, 256))
    s1 = _prop_fs(g1, ei)
    g2 = _tc_layer23(s1, degp, W2, b2.reshape(1, 256), W3p)
    s2p = _prop_es(g2, ei)
    out = _tc_final(s2p, g2, degp, b3.reshape(1, 64))
    return out[:N]


# speedup vs baseline: 26.8894x; 3.1414x over previous
import functools

import jax
import jax.numpy as jnp
from jax import lax
from jax.experimental import pallas as pl
from jax.experimental.pallas import tpu as pltpu
from jax.experimental.pallas import tpu_sc as plsc

N = 10000
NPAD = 10240
E = 320000
CH = 128
NCHUNK = 160
EPAD = 16 * NCHUNK * CH
ROWS = NPAD // 16
SEG = 40
RB = 2048

_mesh = plsc.VectorSubcoreMesh(core_axis_name="c", subcore_axis_name="s")


def _prop_body(table, ei_hbm, out_col, e0, nchunk, sid,
               srcv, dstv, bufa, bufb, acc, sema, semb):
    r0 = sid * ROWS
    pltpu.sync_copy(table.at[pl.ds(r0, ROWS)], acc.at[pl.ds(r0, ROWS)])
    plsc.subcore_barrier()
    for g in range(nchunk // SEG):
        pltpu.sync_copy(ei_hbm.at[0, sid, pl.ds(e0 + g * SEG, SEG)], srcv)
        pltpu.sync_copy(ei_hbm.at[1, sid, pl.ds(e0 + g * SEG, SEG)], dstv)
        pltpu.async_copy(table.at[srcv.at[0]], bufa, sema)
        pltpu.async_copy(table.at[srcv.at[1]], bufb, semb)

        @pl.loop(0, SEG - 2, step=2)
        def _(j):
            pltpu.make_async_copy(table.at[srcv.at[j]], bufa, sema).wait()
            pltpu.sync_copy(bufa, acc.at[dstv.at[j]], add=True)
            pltpu.async_copy(table.at[srcv.at[j + 2]], bufa, sema)
            pltpu.make_async_copy(table.at[srcv.at[j + 1]], bufb, semb).wait()
            pltpu.sync_copy(bufb, acc.at[dstv.at[j + 1]], add=True)
            pltpu.async_copy(table.at[srcv.at[j + 3]], bufb, semb)

        pltpu.make_async_copy(table.at[srcv.at[SEG - 2]], bufa, sema).wait()
        pltpu.sync_copy(bufa, acc.at[dstv.at[SEG - 2]], add=True)
        pltpu.make_async_copy(table.at[srcv.at[SEG - 1]], bufb, semb).wait()
        pltpu.sync_copy(bufb, acc.at[dstv.at[SEG - 1]], add=True)

    plsc.subcore_barrier()
    pltpu.sync_copy(acc.at[pl.ds(r0, ROWS)], out_col.at[pl.ds(r0, ROWS)])


@functools.partial(
    pl.kernel,
    out_type=jax.ShapeDtypeStruct((2, NPAD, 128), jnp.float32),
    mesh=_mesh,
    scratch_types=[
        pltpu.VMEM((SEG, CH), jnp.int32),
        pltpu.VMEM((SEG, CH), jnp.int32),
        pltpu.VMEM((CH, 128), jnp.float32),
        pltpu.VMEM((CH, 128), jnp.float32),
        pltpu.VMEM_SHARED((NPAD, 128), jnp.float32),
        pltpu.SemaphoreType.DMA,
        pltpu.SemaphoreType.DMA,
    ],
    compiler_params=pltpu.CompilerParams(needs_layout_passes=False),
)
def _prop_es(g_hbm, ei_hbm, out_hbm, srcv, dstv, bufa, bufb, acc, sema, semb):
    cid = lax.axis_index("c")
    sid = lax.axis_index("s")
    _prop_body(g_hbm, ei_hbm, out_hbm.at[cid], cid * (NCHUNK // 2), NCHUNK // 2,
               sid, srcv, dstv, bufa, bufb, acc, sema, semb)


@functools.partial(
    pl.kernel,
    out_type=jax.ShapeDtypeStruct((2, NPAD, 128), jnp.float32),
    mesh=_mesh,
    scratch_types=[
        pltpu.VMEM((SEG, CH), jnp.int32),
        pltpu.VMEM((SEG, CH), jnp.int32),
        pltpu.VMEM((CH, 128), jnp.float32),
        pltpu.VMEM((CH, 128), jnp.float32),
        pltpu.VMEM_SHARED((NPAD, 128), jnp.float32),
        pltpu.SemaphoreType.DMA,
        pltpu.SemaphoreType.DMA,
    ],
    compiler_params=pltpu.CompilerParams(needs_layout_passes=False),
)
def _prop_fs(g_hbm, ei_hbm, out_hbm, srcv, dstv, bufa, bufb, acc, sema, semb):
    cid = lax.axis_index("c")
    sid = lax.axis_index("s")
    _prop_body(g_hbm.at[cid], ei_hbm, out_hbm.at[cid], 0, NCHUNK,
               sid, srcv, dstv, bufa, bufb, acc, sema, semb)


@functools.partial(
    pl.kernel,
    out_type=jax.ShapeDtypeStruct((2, NPAD), jnp.float32),
    mesh=_mesh,
    scratch_types=[
        pltpu.VMEM((NCHUNK // 2, CH), jnp.int32),
        pltpu.VMEM((NPAD,), jnp.float32),
        pltpu.VMEM((16, ROWS), jnp.float32),
        pltpu.VMEM_SHARED((16, NPAD), jnp.float32),
    ],
    compiler_params=pltpu.CompilerParams(needs_layout_passes=False),
)
def _sc_degree(ei_hbm, out_hbm, dstv, hist, slab, shared):
    cid = lax.axis_index("c")
    sid = lax.axis_index("s")
    pltpu.sync_copy(ei_hbm.at[1, sid, pl.ds(cid * (NCHUNK // 2), NCHUNK // 2)], dstv)

    @pl.loop(0, NPAD // 16)
    def _(i):
        hist[pl.ds(i * 16, 16)] = jnp.zeros((16,), jnp.float32)

    ones = jnp.ones((16,), jnp.float32)
    lane = lax.iota(jnp.int32, 16)

    @pl.loop(0, NCHUNK // 2)
    def _(j):
        for k in range(CH // 16):
            idx = dstv[j, pl.ds(k * 16, 16)]
            for l in range(16):
                plsc.addupdate_scatter(hist, [idx], ones, mask=lane == l)

    pltpu.sync_copy(hist, shared.at[sid])
    plsc.subcore_barrier()
    pltpu.sync_copy(shared.at[:, pl.ds(sid * ROWS, ROWS)], slab)

    @pl.loop(0, ROWS // 16)
    def _(c):
        acc16 = slab[0, pl.ds(c * 16, 16)]
        for r in range(1, 16):
            acc16 += slab[r, pl.ds(c * 16, 16)]
        hist[pl.ds(c * 16, 16)] = acc16

    pltpu.sync_copy(hist.at[pl.ds(0, ROWS)], out_hbm.at[cid, pl.ds(sid * ROWS, ROWS)])


def _dinv(deg_ref):
    return lax.rsqrt(deg_ref[0] + deg_ref[1] + 1.0)


def _tc_g0(x_pad, degp):
    def body(x_ref, deg_ref, o_ref):
        o_ref[...] = x_ref[...] * _dinv(deg_ref)

    return pl.pallas_call(
        body,
        grid=(NPAD // RB,),
        in_specs=[
            pl.BlockSpec((RB, 128), lambda i: (i, 0)),
            pl.BlockSpec((2, RB, 1), lambda i: (0, i, 0)),
        ],
        out_specs=pl.BlockSpec((RB, 128), lambda i: (i, 0)),
        out_shape=jax.ShapeDtypeStruct((NPAD, 128), jnp.float32),
    )(x_pad, degp)


def _tc_layer1(s0p, g0, degp, W1, b1):
    def body(s_ref, g_ref, deg_ref, w_ref, b_ref, o_ref):
        dinv = _dinv(deg_ref)
        u = (s_ref[0] + s_ref[1] - g_ref[...]) * dinv
        h = jnp.dot(u, w_ref[...], preferred_element_type=jnp.float32)
        g = jax.nn.relu(h + b_ref[...]) * dinv
        o_ref[0] = g[:, :128]
        o_ref[1] = g[:, 128:]

    return pl.pallas_call(
        body,
        grid=(NPAD // RB,),
        in_specs=[
            pl.BlockSpec((2, RB, 128), lambda i: (0, i, 0)),
            pl.BlockSpec((RB, 128), lambda i: (i, 0)),
            pl.BlockSpec((2, RB, 1), lambda i: (0, i, 0)),
            pl.BlockSpec((128, 256), lambda i: (0, 0)),
            pl.BlockSpec((1, 256), lambda i: (0, 0)),
        ],
        out_specs=pl.BlockSpec((2, RB, 128), lambda i: (0, i, 0)),
        out_shape=jax.ShapeDtypeStruct((2, NPAD, 128), jnp.float32),
    )(s0p, g0, degp, W1, b1)


def _tc_layer23(s1, degp, W2, b2, W3p):
    def body(s_ref, deg_ref, w2_ref, b2_ref, w3_ref, o_ref):
        dinv = _dinv(deg_ref)
        h = jnp.dot(s_ref[0] * dinv, w2_ref[:128], preferred_element_type=jnp.float32)
        h += jnp.dot(s_ref[1] * dinv, w2_ref[128:], preferred_element_type=jnp.float32)
        h = jax.nn.relu(h + b2_ref[...])
        o_ref[...] = jnp.dot(h, w3_ref[...], preferred_element_type=jnp.float32) * dinv

    return pl.pallas_call(
        body,
        grid=(NPAD // RB,),
        in_specs=[
            pl.BlockSpec((2, RB, 128), lambda i: (0, i, 0)),
            pl.BlockSpec((2, RB, 1), lambda i: (0, i, 0)),
            pl.BlockSpec((256, 256), lambda i: (0, 0)),
            pl.BlockSpec((1, 256), lambda i: (0, 0)),
            pl.BlockSpec((256, 128), lambda i: (0, 0)),
        ],
        out_specs=pl.BlockSpec((RB, 128), lambda i: (i, 0)),
        out_shape=jax.ShapeDtypeStruct((NPAD, 128), jnp.float32),
    )(s1, degp, W2, b2, W3p)


def _tc_final(s2p, g2, degp, b3):
    def body(s_ref, g_ref, deg_ref, b_ref, o_ref):
        dinv = _dinv(deg_ref)
        s = (s_ref[0] + s_ref[1] - g_ref[...]) * dinv
        o_ref[...] = s[:, :64] + b_ref[...]

    return pl.pallas_call(
        body,
        grid=(NPAD // RB,),
        in_specs=[
            pl.BlockSpec((2, RB, 128), lambda i: (0, i, 0)),
            pl.BlockSpec((RB, 128), lambda i: (i, 0)),
            pl.BlockSpec((2, RB, 1), lambda i: (0, i, 0)),
            pl.BlockSpec((1, 64), lambda i: (0, 0)),
        ],
        out_specs=pl.BlockSpec((RB, 64), lambda i: (i, 0)),
        out_shape=jax.ShapeDtypeStruct((NPAD, 64), jnp.float32),
    )(s2p, g2, degp, b3)


def kernel(features, edge_index, W1, b1, W2, b2, W3, b3):
    x_pad = jnp.pad(features, ((0, NPAD - N), (0, 0)))
    W3p = jnp.pad(W3, ((0, 0), (0, 64)))
    pad_idx = N + jnp.arange(EPAD - E, dtype=jnp.int32) % 128
    src = jnp.concatenate([edge_index[0], pad_idx]).reshape(16, NCHUNK, CH)
    dst = jnp.concatenate([edge_index[1], pad_idx]).reshape(16, NCHUNK, CH)
    ei = jnp.stack([src, dst])

    degp = _sc_degree(ei).reshape(2, NPAD, 1)
    g0 = _tc_g0(x_pad, degp)
    s0p = _prop_es(g0, ei)
    g1 = _tc_layer1(s0p, g0, degp, W1, b1.reshape(1, 256))
    s1 = _prop_fs(g1, ei)
    g2 = _tc_layer23(s1, degp, W2, b2.reshape(1, 256), W3p)
    s2p = _prop_es(g2, ei)
    out = _tc_final(s2p, g2, degp, b3.reshape(1, 64))
    return out[:N]


# speedup vs baseline: 27.0399x; 1.0056x over previous
import functools

import jax
import jax.numpy as jnp
from jax import lax
from jax.experimental import pallas as pl
from jax.experimental.pallas import tpu as pltpu
from jax.experimental.pallas import tpu_sc as plsc

N = 10000
NPAD = 10240
E = 320000
CH = 128
NCHUNK = 160
EPAD = 16 * NCHUNK * CH
ROWS = NPAD // 16
SEG = 40
RB = 2048
FRB = 2000

_mesh = plsc.VectorSubcoreMesh(core_axis_name="c", subcore_axis_name="s")


def _prop_body(table, src_hbm, dst_hbm, out_col, e0, nchunk, sid,
               srcv, dstv, bufa, bufb, acc, sema, semb):
    r0 = sid * ROWS
    pltpu.sync_copy(table.at[pl.ds(r0, ROWS)], acc.at[pl.ds(r0, ROWS)])
    plsc.subcore_barrier()
    for g in range(nchunk // SEG):
        pltpu.sync_copy(src_hbm.at[sid, pl.ds(e0 + g * SEG, SEG)], srcv)
        pltpu.sync_copy(dst_hbm.at[sid, pl.ds(e0 + g * SEG, SEG)], dstv)
        pltpu.async_copy(table.at[srcv.at[0]], bufa, sema)
        pltpu.async_copy(table.at[srcv.at[1]], bufb, semb)

        @pl.loop(0, SEG - 2, step=2)
        def _(j):
            pltpu.make_async_copy(table.at[srcv.at[j]], bufa, sema).wait()
            pltpu.sync_copy(bufa, acc.at[dstv.at[j]], add=True)
            pltpu.async_copy(table.at[srcv.at[j + 2]], bufa, sema)
            pltpu.make_async_copy(table.at[srcv.at[j + 1]], bufb, semb).wait()
            pltpu.sync_copy(bufb, acc.at[dstv.at[j + 1]], add=True)
            pltpu.async_copy(table.at[srcv.at[j + 3]], bufb, semb)

        pltpu.make_async_copy(table.at[srcv.at[SEG - 2]], bufa, sema).wait()
        pltpu.sync_copy(bufa, acc.at[dstv.at[SEG - 2]], add=True)
        pltpu.make_async_copy(table.at[srcv.at[SEG - 1]], bufb, semb).wait()
        pltpu.sync_copy(bufb, acc.at[dstv.at[SEG - 1]], add=True)

    plsc.subcore_barrier()
    pltpu.sync_copy(acc.at[pl.ds(r0, ROWS)], out_col.at[pl.ds(r0, ROWS)])


@functools.partial(
    pl.kernel,
    out_type=jax.ShapeDtypeStruct((2, NPAD, 128), jnp.float32),
    mesh=_mesh,
    scratch_types=[
        pltpu.VMEM((SEG, CH), jnp.int32),
        pltpu.VMEM((SEG, CH), jnp.int32),
        pltpu.VMEM((CH, 128), jnp.float32),
        pltpu.VMEM((CH, 128), jnp.float32),
        pltpu.VMEM_SHARED((NPAD, 128), jnp.float32),
        pltpu.SemaphoreType.DMA,
        pltpu.SemaphoreType.DMA,
    ],
    compiler_params=pltpu.CompilerParams(needs_layout_passes=False),
)
def _prop_es(g_hbm, src_hbm, dst_hbm, out_hbm,
             srcv, dstv, bufa, bufb, acc, sema, semb):
    cid = lax.axis_index("c")
    sid = lax.axis_index("s")
    _prop_body(g_hbm, src_hbm, dst_hbm, out_hbm.at[cid],
               cid * (NCHUNK // 2), NCHUNK // 2,
               sid, srcv, dstv, bufa, bufb, acc, sema, semb)


@functools.partial(
    pl.kernel,
    out_type=jax.ShapeDtypeStruct((2, NPAD, 128), jnp.float32),
    mesh=_mesh,
    scratch_types=[
        pltpu.VMEM((SEG, CH), jnp.int32),
        pltpu.VMEM((SEG, CH), jnp.int32),
        pltpu.VMEM((CH, 128), jnp.float32),
        pltpu.VMEM((CH, 128), jnp.float32),
        pltpu.VMEM_SHARED((NPAD, 128), jnp.float32),
        pltpu.SemaphoreType.DMA,
        pltpu.SemaphoreType.DMA,
    ],
    compiler_params=pltpu.CompilerParams(needs_layout_passes=False),
)
def _prop_fs(g_hbm, src_hbm, dst_hbm, out_hbm,
             srcv, dstv, bufa, bufb, acc, sema, semb):
    cid = lax.axis_index("c")
    sid = lax.axis_index("s")
    _prop_body(g_hbm.at[cid], src_hbm, dst_hbm, out_hbm.at[cid], 0, NCHUNK,
               sid, srcv, dstv, bufa, bufb, acc, sema, semb)


@functools.partial(
    pl.kernel,
    out_type=jax.ShapeDtypeStruct((2, NPAD), jnp.float32),
    mesh=_mesh,
    scratch_types=[
        pltpu.VMEM((NCHUNK // 2, CH), jnp.int32),
        pltpu.VMEM((NPAD,), jnp.float32),
        pltpu.VMEM((16, ROWS), jnp.float32),
        pltpu.VMEM_SHARED((16, NPAD), jnp.float32),
    ],
    compiler_params=pltpu.CompilerParams(needs_layout_passes=False),
)
def _sc_degree(dst_hbm, out_hbm, dstv, hist, slab, shared):
    cid = lax.axis_index("c")
    sid = lax.axis_index("s")
    pltpu.sync_copy(dst_hbm.at[sid, pl.ds(cid * (NCHUNK // 2), NCHUNK // 2)], dstv)

    @pl.loop(0, NPAD // 16)
    def _(i):
        hist[pl.ds(i * 16, 16)] = jnp.zeros((16,), jnp.float32)

    ones = jnp.ones((16,), jnp.float32)
    lane = lax.iota(jnp.int32, 16)

    @pl.loop(0, NCHUNK // 2)
    def _(j):
        for k in range(CH // 16):
            idx = dstv[j, pl.ds(k * 16, 16)]
            for l in range(16):
                plsc.addupdate_scatter(hist, [idx], ones, mask=lane == l)

    pltpu.sync_copy(hist, shared.at[sid])
    plsc.subcore_barrier()
    pltpu.sync_copy(shared.at[:, pl.ds(sid * ROWS, ROWS)], slab)

    @pl.loop(0, ROWS // 16)
    def _(c):
        acc16 = slab[0, pl.ds(c * 16, 16)]
        for r in range(1, 16):
            acc16 += slab[r, pl.ds(c * 16, 16)]
        hist[pl.ds(c * 16, 16)] = acc16

    pltpu.sync_copy(hist.at[pl.ds(0, ROWS)], out_hbm.at[cid, pl.ds(sid * ROWS, ROWS)])


def _dinv(deg_ref):
    return lax.rsqrt(deg_ref[0] + deg_ref[1] + 1.0)


def _tc_g0(x_pad, degp):
    def body(x_ref, deg_ref, o_ref):
        o_ref[...] = x_ref[...] * _dinv(deg_ref)

    return pl.pallas_call(
        body,
        grid=(NPAD // RB,),
        in_specs=[
            pl.BlockSpec((RB, 128), lambda i: (i, 0)),
            pl.BlockSpec((2, RB, 1), lambda i: (0, i, 0)),
        ],
        out_specs=pl.BlockSpec((RB, 128), lambda i: (i, 0)),
        out_shape=jax.ShapeDtypeStruct((NPAD, 128), jnp.float32),
    )(x_pad, degp)


def _tc_layer1(s0p, g0, degp, W1, b1):
    def body(s_ref, g_ref, deg_ref, w_ref, b_ref, o_ref):
        dinv = _dinv(deg_ref)
        u = (s_ref[0] + s_ref[1] - g_ref[...]) * dinv
        h = jnp.dot(u, w_ref[...], preferred_element_type=jnp.float32)
        g = jax.nn.relu(h + b_ref[...]) * dinv
        o_ref[0] = g[:, :128]
        o_ref[1] = g[:, 128:]

    return pl.pallas_call(
        body,
        grid=(NPAD // RB,),
        in_specs=[
            pl.BlockSpec((2, RB, 128), lambda i: (0, i, 0)),
            pl.BlockSpec((RB, 128), lambda i: (i, 0)),
            pl.BlockSpec((2, RB, 1), lambda i: (0, i, 0)),
            pl.BlockSpec((128, 256), lambda i: (0, 0)),
            pl.BlockSpec((1, 256), lambda i: (0, 0)),
        ],
        out_specs=pl.BlockSpec((2, RB, 128), lambda i: (0, i, 0)),
        out_shape=jax.ShapeDtypeStruct((2, NPAD, 128), jnp.float32),
    )(s0p, g0, degp, W1, b1)


def _tc_layer23(s1, degp, W2, b2, W3p):
    def body(s_ref, deg_ref, w2_ref, b2_ref, w3_ref, o_ref):
        dinv = _dinv(deg_ref)
        h = jnp.dot(s_ref[0] * dinv, w2_ref[:128], preferred_element_type=jnp.float32)
        h += jnp.dot(s_ref[1] * dinv, w2_ref[128:], preferred_element_type=jnp.float32)
        h = jax.nn.relu(h + b2_ref[...])
        o_ref[...] = jnp.dot(h, w3_ref[...], preferred_element_type=jnp.float32) * dinv

    return pl.pallas_call(
        body,
        grid=(NPAD // RB,),
        in_specs=[
            pl.BlockSpec((2, RB, 128), lambda i: (0, i, 0)),
            pl.BlockSpec((2, RB, 1), lambda i: (0, i, 0)),
            pl.BlockSpec((256, 256), lambda i: (0, 0)),
            pl.BlockSpec((1, 256), lambda i: (0, 0)),
            pl.BlockSpec((256, 128), lambda i: (0, 0)),
        ],
        out_specs=pl.BlockSpec((RB, 128), lambda i: (i, 0)),
        out_shape=jax.ShapeDtypeStruct((NPAD, 128), jnp.float32),
    )(s1, degp, W2, b2, W3p)


def _tc_final(s2p, g2, degp, b3):
    def body(s_ref, g_ref, deg_ref, b_ref, o_ref):
        dinv = _dinv(deg_ref)
        s = (s_ref[0] + s_ref[1] - g_ref[...]) * dinv
        o_ref[...] = s[:, :64] + b_ref[...]

    return pl.pallas_call(
        body,
        grid=(N // FRB,),
        in_specs=[
            pl.BlockSpec((2, FRB, 128), lambda i: (0, i, 0)),
            pl.BlockSpec((FRB, 128), lambda i: (i, 0)),
            pl.BlockSpec((2, FRB, 1), lambda i: (0, i, 0)),
            pl.BlockSpec((1, 64), lambda i: (0, 0)),
        ],
        out_specs=pl.BlockSpec((FRB, 64), lambda i: (i, 0)),
        out_shape=jax.ShapeDtypeStruct((N, 64), jnp.float32),
    )(s2p, g2, degp, b3)


def kernel(features, edge_index, W1, b1, W2, b2, W3, b3):
    x_pad = jnp.pad(features, ((0, NPAD - N), (0, 0)))
    W3p = jnp.pad(W3, ((0, 0), (0, 64)))
    pad_idx = N + jnp.arange(EPAD - E, dtype=jnp.int32) % 128
    src = jnp.concatenate([edge_index[0], pad_idx]).reshape(16, NCHUNK, CH)
    dst = jnp.concatenate([edge_index[1], pad_idx]).reshape(16, NCHUNK, CH)

    degp = _sc_degree(dst).reshape(2, NPAD, 1)
    g0 = _tc_g0(x_pad, degp)
    s0p = _prop_es(g0, src, dst)
    g1 = _tc_layer1(s0p, g0, degp, W1, b1.reshape(1, 256))
    s1 = _prop_fs(g1, src, dst)
    g2 = _tc_layer23(s1, degp, W2, b2.reshape(1, 256), W3p)
    s2p = _prop_es(g2, src, dst)
    return _tc_final(s2p, g2, degp, b3.reshape(1, 64))
